# baseline (speedup 1.0000x reference)
import functools

import jax
import jax.numpy as jnp
from jax import lax
from jax.experimental import pallas as pl
from jax.experimental.pallas import tpu as pltpu
from jax.experimental.pallas import tpu_sc as plsc

_HIGHEST = jax.lax.Precision.HIGHEST


def _dot(a, b):
    return jax.lax.dot_general(a, b, (((1,), (0,)), ((), ())),
                               precision=_HIGHEST,
                               preferred_element_type=jnp.float32)


def _fold_body(w1_ref, b1_ref, w2_ref, b2_ref, gws_ref, gwd_ref, gas_ref,
               gad_ref, gb_ref, w3_ref, b3_ref, w5_ref, b5_ref,
               uw_ref, cw_ref, us_ref, cs_ref):
    w1 = w1_ref[...]
    w2 = w2_ref[...]
    gws = gws_ref[...]
    w3 = w3_ref[...]
    w5 = w5_ref[...]
    b1 = b1_ref[...][None, :]
    b2 = b2_ref[...][None, :]
    v = _dot(w3, w5)
    ga = _dot(gws, gas_ref[...][:, None])
    gw = _dot(gws, v)
    gad = _dot(gwd_ref[...], gad_ref[...][:, None])
    z6 = jnp.zeros((128, 6), jnp.float32)
    uw_ref[...] = jnp.concatenate([_dot(w1, ga), _dot(w1, gw), z6], axis=1)
    us_ref[...] = jnp.concatenate([_dot(w2, gad), _dot(w2, v), z6], axis=1)
    c0 = (_dot(gb_ref[...][None, :], v) + _dot(b3_ref[...][None, :], w5)
          + b5_ref[...][None, :])
    z1 = jnp.zeros((1, 6), jnp.float32)
    cw_ref[...] = jnp.concatenate([_dot(b1, ga), _dot(b1, gw), z1], axis=1)
    cs_ref[...] = jnp.concatenate([_dot(b2, gad), _dot(b2, v) + c0, z1],
                                  axis=1)


def _fold_weights(W1, b1, W2, b2, g2_Ws, g2_Wd, g2_as, g2_ad, g2_b,
                  W3, b3, W5, b5):
    o = jax.ShapeDtypeStruct
    return pl.pallas_call(
        _fold_body,
        out_shape=(o((128, 8), jnp.float32), o((1, 8), jnp.float32),
                   o((128, 8), jnp.float32), o((1, 8), jnp.float32)),
    )(W1, b1, W2, b2, g2_Ws, g2_Wd, g2_as, g2_ad, g2_b, W3, b3, W5, b5)


_BLK = 10000


def _mm_body(x_ref, u_ref, c_ref, o1_ref, o2_ref):
    y = _dot(x_ref[...], u_ref[...]) + c_ref[...]
    o1_ref[...] = y[:, 0][None, None, :]
    o2_ref[...] = y[:, 1][None, None, :]


def _project(x, u, c):
    n = x.shape[0]
    g = n // _BLK
    o = jax.ShapeDtypeStruct
    return pl.pallas_call(
        _mm_body,
        grid=(g,),
        in_specs=[
            pl.BlockSpec((_BLK, 128), lambda i: (i, 0)),
            pl.BlockSpec((128, 8), lambda i: (0, 0)),
            pl.BlockSpec((1, 8), lambda i: (0, 0)),
        ],
        out_specs=(pl.BlockSpec((1, 1, _BLK), lambda i: (i, 0, 0)),
                   pl.BlockSpec((1, 1, _BLK), lambda i: (i, 0, 0))),
        out_shape=(o((g, 1, _BLK), jnp.float32),
                   o((g, 1, _BLK), jnp.float32)),
    )(x, u, c)


_NTILES = 32
_LANES = 16


def _sc_body(ns, nw, ne, src_hbm, dst_hbm, aw_hbm, wt_hbm, as_hbm, out_hbm,
             tbl_ref, tbls_ref, acc_ref, exb_ref, srcb_ref, dstb_ref):
    nept = ne // _NTILES
    wid = lax.axis_index("s") * 2 + lax.axis_index("c")
    base = wid * nept
    pltpu.sync_copy(src_hbm.at[pl.ds(base, nept)], srcb_ref)
    pltpu.sync_copy(dst_hbm.at[pl.ds(base, nept)], dstb_ref)
    pltpu.sync_copy(aw_hbm, tbl_ref)
    pltpu.sync_copy(as_hbm, tbls_ref)

    def zero_body(i, _):
        acc_ref[pl.ds(i * _LANES, _LANES)] = jnp.zeros((_LANES,), jnp.float32)
        return 0

    lax.fori_loop(0, ns // _LANES, zero_body, 0)

    zero16 = jnp.zeros((_LANES,), jnp.int32)

    def phase_a(i, _):
        sl = pl.ds(i * _LANES, _LANES)
        sv = srcb_ref[sl]
        dv = dstb_ref[sl]
        rv = lax.div(sv, _BLK)
        cv = sv - rv * _BLK
        a1 = plsc.load_gather(tbl_ref, [rv, zero16, cv])
        a2 = plsc.load_gather(tbls_ref, [zero16, zero16, dv])
        al = a1 + a2
        al = jnp.where(al > 0, al, al * jnp.float32(0.2))
        ex = jnp.exp(al)
        ex = jnp.where(sv != dv, ex, jnp.float32(0.0))
        exb_ref[sl] = ex
        plsc.addupdate_scatter(acc_ref, [dv], ex)
        return 0

    lax.fori_loop(0, nept // _LANES, phase_a, 0)
    pltpu.sync_copy(acc_ref, out_hbm.at[wid])

    pltpu.sync_copy(wt_hbm, tbl_ref)
    lax.fori_loop(0, ns // _LANES, zero_body, 0)

    def phase_b(i, _):
        sl = pl.ds(i * _LANES, _LANES)
        sv = srcb_ref[sl]
        dv = dstb_ref[sl]
        rv = lax.div(sv, _BLK)
        cv = sv - rv * _BLK
        wv = plsc.load_gather(tbl_ref, [rv, zero16, cv])
        plsc.addupdate_scatter(acc_ref, [dv], exb_ref[sl] * wv)
        return 0

    lax.fori_loop(0, nept // _LANES, phase_b, 0)
    pltpu.sync_copy(acc_ref, out_hbm.at[_NTILES + wid])


def _edge_softmax(src, dst, a_word, w_word, a_sent):
    ns = a_sent.shape[2]
    nw_rows = a_word.shape[0]
    ne = src.shape[0]
    nept = ne // _NTILES
    mesh = plsc.VectorSubcoreMesh(core_axis_name="c", subcore_axis_name="s",
                                  num_cores=2, num_subcores=16)
    return pl.kernel(
        functools.partial(_sc_body, ns, nw_rows, ne),
        out_type=jax.ShapeDtypeStruct((2 * _NTILES, ns), jnp.float32),
        mesh=mesh,
        compiler_params=pltpu.CompilerParams(needs_layout_passes=False),
        scratch_types=[
            pltpu.VMEM((nw_rows, 1, _BLK), jnp.float32),
            pltpu.VMEM((1, 1, ns), jnp.float32),
            pltpu.VMEM((ns,), jnp.float32),
            pltpu.VMEM((nept,), jnp.float32),
            pltpu.VMEM((nept,), jnp.int32),
            pltpu.VMEM((nept,), jnp.int32),
        ],
    )(src, dst, a_word, w_word, a_sent)


def _fin_body(parts_ref, a_ref, w_ref, ad_ref, hv_ref, o_ref):
    p = parts_ref[...]
    den = jnp.sum(p[:_NTILES], axis=0)
    num = jnp.sum(p[_NTILES:], axis=0)
    al = a_ref[0, 0, :] + ad_ref[0, 0, :]
    al = jnp.where(al > 0, al, al * jnp.float32(0.2))
    exs = jnp.exp(al)
    res = ((num + exs * w_ref[0, 0, :]) / (den + exs + jnp.float32(1e-16))
           + hv_ref[0, 0, :])
    o_ref[...] = res[:, None]


def _finalize(parts, a_word, w_word, a_sent, hsv):
    ns = a_sent.shape[2]
    return pl.pallas_call(
        _fin_body,
        grid=(1,),
        in_specs=[
            pl.BlockSpec((2 * _NTILES, ns), lambda i: (0, 0)),
            pl.BlockSpec((1, 1, ns), lambda i: (0, 0, 0)),
            pl.BlockSpec((1, 1, ns), lambda i: (0, 0, 0)),
            pl.BlockSpec((1, 1, ns), lambda i: (0, 0, 0)),
            pl.BlockSpec((1, 1, ns), lambda i: (0, 0, 0)),
        ],
        out_specs=pl.BlockSpec((ns, 1), lambda i: (0, 0)),
        out_shape=jax.ShapeDtypeStruct((ns, 1), jnp.float32),
    )(parts, a_word, w_word, a_sent, hsv)


def kernel(Xw, Xs, E, Erev, W1, b1, W2, b2,
           g1_Ws, g1_Wd, g1_as, g1_ad, g1_b,
           g2_Ws, g2_Wd, g2_as, g2_ad, g2_b,
           W3, b3, W4, b4, W5, b5):
    uw, cw, us, cs = _fold_weights(W1, b1, W2, b2, g2_Ws, g2_Wd, g2_as,
                                   g2_ad, g2_b, W3, b3, W5, b5)
    a_word, w_word = _project(Xw, uw, cw)
    a_sent, hsv = _project(Xs, us, cs)
    parts = _edge_softmax(E[1], E[0], a_word, w_word, a_sent)
    return _finalize(parts, a_word, w_word, a_sent, hsv)


# speedup vs baseline: 1.4065x; 1.4065x over previous
import functools

import jax
import jax.numpy as jnp
from jax import lax
from jax.experimental import pallas as pl
from jax.experimental.pallas import tpu as pltpu
from jax.experimental.pallas import tpu_sc as plsc

_HIGHEST = jax.lax.Precision.HIGHEST


def _dot(a, b):
    return jax.lax.dot_general(a, b, (((1,), (0,)), ((), ())),
                               precision=_HIGHEST,
                               preferred_element_type=jnp.float32)


def _fold_body(w1_ref, b1_ref, w2_ref, b2_ref, gws_ref, gwd_ref, gas_ref,
               gad_ref, gb_ref, w3_ref, b3_ref, w5_ref, b5_ref,
               uw_ref, cw_ref, us_ref, cs_ref):
    w1 = w1_ref[...]
    w2 = w2_ref[...]
    gws = gws_ref[...]
    w3 = w3_ref[...]
    w5 = w5_ref[...]
    b1 = b1_ref[...][None, :]
    b2 = b2_ref[...][None, :]
    v = _dot(w3, w5)
    ga = _dot(gws, gas_ref[...][:, None])
    gw = _dot(gws, v)
    gad = _dot(gwd_ref[...], gad_ref[...][:, None])
    z6 = jnp.zeros((128, 6), jnp.float32)
    uw_ref[...] = jnp.concatenate([_dot(w1, ga), _dot(w1, gw), z6], axis=1)
    us_ref[...] = jnp.concatenate([_dot(w2, gad), _dot(w2, v), z6], axis=1)
    c0 = (_dot(gb_ref[...][None, :], v) + _dot(b3_ref[...][None, :], w5)
          + b5_ref[...][None, :])
    z1 = jnp.zeros((1, 6), jnp.float32)
    cw_ref[...] = jnp.concatenate([_dot(b1, ga), _dot(b1, gw), z1], axis=1)
    cs_ref[...] = jnp.concatenate([_dot(b2, gad), _dot(b2, v) + c0, z1],
                                  axis=1)


def _fold_weights(W1, b1, W2, b2, g2_Ws, g2_Wd, g2_as, g2_ad, g2_b,
                  W3, b3, W5, b5):
    o = jax.ShapeDtypeStruct
    return pl.pallas_call(
        _fold_body,
        out_shape=(o((128, 8), jnp.float32), o((1, 8), jnp.float32),
                   o((128, 8), jnp.float32), o((1, 8), jnp.float32)),
    )(W1, b1, W2, b2, g2_Ws, g2_Wd, g2_as, g2_ad, g2_b, W3, b3, W5, b5)


_BLK = 10240


def _mm_body(x_ref, u_ref, c_ref, o1_ref, o2_ref):
    y = _dot(x_ref[...], u_ref[...]) + c_ref[...]
    yt = jnp.transpose(y)
    o1_ref[...] = yt[0]
    o2_ref[...] = yt[1]


def _project(x, u, c):
    n = x.shape[0]
    g = n // _BLK
    o = jax.ShapeDtypeStruct
    return pl.pallas_call(
        _mm_body,
        grid=(g,),
        in_specs=[
            pl.BlockSpec((_BLK, 128), lambda i: (i, 0)),
            pl.BlockSpec((128, 8), lambda i: (0, 0)),
            pl.BlockSpec((1, 8), lambda i: (0, 0)),
        ],
        out_specs=(pl.BlockSpec((_BLK,), lambda i: (i,)),
                   pl.BlockSpec((_BLK,), lambda i: (i,))),
        out_shape=(o((n,), jnp.float32), o((n,), jnp.float32)),
    )(x, u, c)


_NTILES = 32
_LANES = 16


def _sc_body(ns, nw, ne, src_hbm, dst_hbm, aw_hbm, wt_hbm, as_hbm, out_hbm,
             tbl_ref, tbls_ref, acc_ref, exb_ref, srcb_ref, dstb_ref):
    nept = ne // _NTILES
    wid = lax.axis_index("s") * 2 + lax.axis_index("c")
    base = wid * nept
    pltpu.sync_copy(src_hbm.at[pl.ds(base, nept)], srcb_ref)
    pltpu.sync_copy(dst_hbm.at[pl.ds(base, nept)], dstb_ref)
    pltpu.sync_copy(aw_hbm, tbl_ref)
    pltpu.sync_copy(as_hbm, tbls_ref)

    def zero_body(i, _):
        acc_ref[pl.ds(i * _LANES, _LANES)] = jnp.zeros((_LANES,), jnp.float32)
        return 0

    lax.fori_loop(0, ns // _LANES, zero_body, 0)

    def phase_a(i, _):
        sl = pl.ds(i * _LANES, _LANES)
        sv = srcb_ref[sl]
        dv = dstb_ref[sl]
        a1 = plsc.load_gather(tbl_ref, [sv])
        a2 = plsc.load_gather(tbls_ref, [dv])
        al = a1 + a2
        al = jnp.where(al > 0, al, al * jnp.float32(0.2))
        ex = jnp.exp(al)
        ex = jnp.where(sv != dv, ex, jnp.float32(0.0))
        exb_ref[sl] = ex
        plsc.addupdate_scatter(acc_ref, [dv], ex)
        return 0

    lax.fori_loop(0, nept // _LANES, phase_a, 0)
    pltpu.sync_copy(acc_ref, out_hbm.at[wid])

    pltpu.sync_copy(wt_hbm, tbl_ref)
    lax.fori_loop(0, ns // _LANES, zero_body, 0)

    def phase_b(i, _):
        sl = pl.ds(i * _LANES, _LANES)
        sv = srcb_ref[sl]
        dv = dstb_ref[sl]
        wv = plsc.load_gather(tbl_ref, [sv])
        plsc.addupdate_scatter(acc_ref, [dv], exb_ref[sl] * wv)
        return 0

    lax.fori_loop(0, nept // _LANES, phase_b, 0)
    pltpu.sync_copy(acc_ref, out_hbm.at[_NTILES + wid])


def _edge_softmax(ns, src, dst, a_word, w_word, a_sent):
    nw_pad = a_word.shape[0]
    ne = src.shape[0]
    nept = ne // _NTILES
    mesh = plsc.VectorSubcoreMesh(core_axis_name="c", subcore_axis_name="s",
                                  num_cores=2, num_subcores=16)
    return pl.kernel(
        functools.partial(_sc_body, ns, nw_pad, ne),
        out_type=jax.ShapeDtypeStruct((2 * _NTILES, ns), jnp.float32),
        mesh=mesh,
        compiler_params=pltpu.CompilerParams(needs_layout_passes=False),
        scratch_types=[
            pltpu.VMEM((nw_pad,), jnp.float32),
            pltpu.VMEM((a_sent.shape[0],), jnp.float32),
            pltpu.VMEM((ns,), jnp.float32),
            pltpu.VMEM((nept,), jnp.float32),
            pltpu.VMEM((nept,), jnp.int32),
            pltpu.VMEM((nept,), jnp.int32),
        ],
    )(src, dst, a_word, w_word, a_sent)


def _fin_body(ns, parts_ref, a_ref, w_ref, ad_ref, hv_ref, o_ref):
    p = parts_ref[...]
    den = jnp.sum(p[:_NTILES], axis=0)
    num = jnp.sum(p[_NTILES:], axis=0)
    al = a_ref[...][:ns] + ad_ref[...][:ns]
    al = jnp.where(al > 0, al, al * jnp.float32(0.2))
    exs = jnp.exp(al)
    res = ((num + exs * w_ref[...][:ns]) / (den + exs + jnp.float32(1e-16))
           + hv_ref[...][:ns])
    o_ref[...] = res[:, None]


def _finalize(ns, parts, a_word, w_word, a_sent, hsv):
    npad = a_sent.shape[0]
    return pl.pallas_call(
        functools.partial(_fin_body, ns),
        grid=(1,),
        in_specs=[
            pl.BlockSpec((2 * _NTILES, ns), lambda i: (0, 0)),
            pl.BlockSpec((_BLK,), lambda i: (0,)),
            pl.BlockSpec((_BLK,), lambda i: (0,)),
            pl.BlockSpec((npad,), lambda i: (0,)),
            pl.BlockSpec((npad,), lambda i: (0,)),
        ],
        out_specs=pl.BlockSpec((ns, 1), lambda i: (0, 0)),
        out_shape=jax.ShapeDtypeStruct((ns, 1), jnp.float32),
    )(parts, a_word, w_word, a_sent, hsv)


def kernel(Xw, Xs, E, Erev, W1, b1, W2, b2,
           g1_Ws, g1_Wd, g1_as, g1_ad, g1_b,
           g2_Ws, g2_Wd, g2_as, g2_ad, g2_b,
           W3, b3, W4, b4, W5, b5):
    uw, cw, us, cs = _fold_weights(W1, b1, W2, b2, g2_Ws, g2_Wd, g2_as,
                                   g2_ad, g2_b, W3, b3, W5, b5)
    ns = Xs.shape[0]
    pw = -Xw.shape[0] % _BLK
    ps = -ns % _BLK
    Xwp = jnp.pad(Xw, ((0, pw), (0, 0)))
    Xsp = jnp.pad(Xs, ((0, ps), (0, 0)))
    a_word, w_word = _project(Xwp, uw, cw)
    a_sent, hsv = _project(Xsp, us, cs)
    parts = _edge_softmax(ns, E[1], E[0], a_word, w_word, a_sent)
    return _finalize(ns, parts, a_word, w_word, a_sent, hsv)


# speedup vs baseline: 1.7187x; 1.2220x over previous
import functools

import jax
import jax.numpy as jnp
from jax import lax
from jax.experimental import pallas as pl
from jax.experimental.pallas import tpu as pltpu
from jax.experimental.pallas import tpu_sc as plsc

_HIGHEST = jax.lax.Precision.HIGHEST


def _dot(a, b):
    return jax.lax.dot_general(a, b, (((1,), (0,)), ((), ())),
                               precision=_HIGHEST,
                               preferred_element_type=jnp.float32)


def _fold_body(w1_ref, b1_ref, w2_ref, b2_ref, gws_ref, gwd_ref, gas_ref,
               gad_ref, gb_ref, w3_ref, b3_ref, w5_ref, b5_ref,
               uw_ref, cw_ref, us_ref, cs_ref):
    w1 = w1_ref[...]
    w2 = w2_ref[...]
    gws = gws_ref[...]
    w3 = w3_ref[...]
    w5 = w5_ref[...]
    b1 = b1_ref[...][None, :]
    b2 = b2_ref[...][None, :]
    v = _dot(w3, w5)
    ga = _dot(gws, gas_ref[...][:, None])
    gw = _dot(gws, v)
    gad = _dot(gwd_ref[...], gad_ref[...][:, None])
    z6 = jnp.zeros((128, 6), jnp.float32)
    uw_ref[...] = jnp.concatenate([_dot(w1, ga), _dot(w1, gw), z6], axis=1)
    us_ref[...] = jnp.concatenate([_dot(w2, gad), _dot(w2, v), z6], axis=1)
    c0 = (_dot(gb_ref[...][None, :], v) + _dot(b3_ref[...][None, :], w5)
          + b5_ref[...][None, :])
    z1 = jnp.zeros((1, 6), jnp.float32)
    cw_ref[...] = jnp.concatenate([_dot(b1, ga), _dot(b1, gw), z1], axis=1)
    cs_ref[...] = jnp.concatenate([_dot(b2, gad), _dot(b2, v) + c0, z1],
                                  axis=1)


def _fold_weights(W1, b1, W2, b2, g2_Ws, g2_Wd, g2_as, g2_ad, g2_b,
                  W3, b3, W5, b5):
    o = jax.ShapeDtypeStruct
    return pl.pallas_call(
        _fold_body,
        out_shape=(o((128, 8), jnp.float32), o((1, 8), jnp.float32),
                   o((128, 8), jnp.float32), o((1, 8), jnp.float32)),
    )(W1, b1, W2, b2, g2_Ws, g2_Wd, g2_as, g2_ad, g2_b, W3, b3, W5, b5)


_BLK = 10240


def _mm_body(x_ref, u_ref, c_ref, o1_ref, o2_ref):
    y = _dot(x_ref[...], u_ref[...]) + c_ref[...]
    yt = jnp.transpose(y[:, :2])
    o1_ref[...] = yt[0]
    o2_ref[...] = yt[1]


def _project(x, u, c):
    n = x.shape[0]
    g = pl.cdiv(n, _BLK)
    npad = g * _BLK
    o = jax.ShapeDtypeStruct
    return pl.pallas_call(
        _mm_body,
        grid=(g,),
        in_specs=[
            pl.BlockSpec((_BLK, 128), lambda i: (i, 0)),
            pl.BlockSpec((128, 8), lambda i: (0, 0)),
            pl.BlockSpec((1, 8), lambda i: (0, 0)),
        ],
        out_specs=(pl.BlockSpec((_BLK,), lambda i: (i,)),
                   pl.BlockSpec((_BLK,), lambda i: (i,))),
        out_shape=(o((npad,), jnp.float32), o((npad,), jnp.float32)),
    )(x, u, c)


_NTILES = 32
_LANES = 16


_UNROLL = 4


def _sc_body(ns, nw, ne, src_hbm, dst_hbm, aw_hbm, wt_hbm, as_hbm, out_hbm,
             tbl_ref, tbls_ref, acc_ref, exb_ref, srcb_ref, dstb_ref,
             sem1, sem2, sem3, sem4):
    nept = ne // _NTILES
    wid = lax.axis_index("s") * 2 + lax.axis_index("c")
    base = wid * nept
    h1 = pltpu.async_copy(src_hbm.at[pl.ds(base, nept)], srcb_ref, sem1)
    h2 = pltpu.async_copy(dst_hbm.at[pl.ds(base, nept)], dstb_ref, sem2)
    h3 = pltpu.async_copy(aw_hbm, tbl_ref, sem3)
    h4 = pltpu.async_copy(as_hbm, tbls_ref, sem4)

    def zero_body(i, _):
        acc_ref[pl.ds(i * _LANES, _LANES)] = jnp.zeros((_LANES,), jnp.float32)
        return 0

    lax.fori_loop(0, ns // _LANES, zero_body, 0)
    h1.wait()
    h2.wait()
    h3.wait()
    h4.wait()

    def phase_a(i, _):
        for k in range(_UNROLL):
            sl = pl.ds((i * _UNROLL + k) * _LANES, _LANES)
            sv = srcb_ref[sl]
            dv = dstb_ref[sl]
            a1 = plsc.load_gather(tbl_ref, [sv])
            a2 = plsc.load_gather(tbls_ref, [dv])
            al = a1 + a2
            al = jnp.where(al > 0, al, al * jnp.float32(0.2))
            ex = jnp.exp(al)
            ex = jnp.where(sv != dv, ex, jnp.float32(0.0))
            exb_ref[sl] = ex
            plsc.addupdate_scatter(acc_ref, [dv], ex)
        return 0

    lax.fori_loop(0, nept // (_LANES * _UNROLL), phase_a, 0)
    hw = pltpu.async_copy(wt_hbm, tbl_ref, sem3)
    pltpu.sync_copy(acc_ref, out_hbm.at[wid])
    lax.fori_loop(0, ns // _LANES, zero_body, 0)
    hw.wait()

    def phase_b(i, _):
        for k in range(_UNROLL):
            sl = pl.ds((i * _UNROLL + k) * _LANES, _LANES)
            sv = srcb_ref[sl]
            dv = dstb_ref[sl]
            wv = plsc.load_gather(tbl_ref, [sv])
            plsc.addupdate_scatter(acc_ref, [dv], exb_ref[sl] * wv)
        return 0

    lax.fori_loop(0, nept // (_LANES * _UNROLL), phase_b, 0)
    pltpu.sync_copy(acc_ref, out_hbm.at[_NTILES + wid])


def _edge_softmax(ns, src, dst, a_word, w_word, a_sent):
    nw_pad = a_word.shape[0]
    ne = src.shape[0]
    nept = ne // _NTILES
    mesh = plsc.VectorSubcoreMesh(core_axis_name="c", subcore_axis_name="s",
                                  num_cores=2, num_subcores=16)
    return pl.kernel(
        functools.partial(_sc_body, ns, nw_pad, ne),
        out_type=jax.ShapeDtypeStruct((2 * _NTILES, ns), jnp.float32),
        mesh=mesh,
        compiler_params=pltpu.CompilerParams(needs_layout_passes=False),
        scratch_types=[
            pltpu.VMEM((nw_pad,), jnp.float32),
            pltpu.VMEM((a_sent.shape[0],), jnp.float32),
            pltpu.VMEM((ns,), jnp.float32),
            pltpu.VMEM((nept,), jnp.float32),
            pltpu.VMEM((nept,), jnp.int32),
            pltpu.VMEM((nept,), jnp.int32),
            pltpu.SemaphoreType.DMA,
            pltpu.SemaphoreType.DMA,
            pltpu.SemaphoreType.DMA,
            pltpu.SemaphoreType.DMA,
        ],
    )(src, dst, a_word, w_word, a_sent)


def _fin_body(ns, parts_ref, a_ref, w_ref, ad_ref, hv_ref, o_ref):
    p = parts_ref[...]
    den = jnp.sum(p[:_NTILES], axis=0)
    num = jnp.sum(p[_NTILES:], axis=0)
    al = a_ref[...][:ns] + ad_ref[...][:ns]
    al = jnp.where(al > 0, al, al * jnp.float32(0.2))
    exs = jnp.exp(al)
    res = ((num + exs * w_ref[...][:ns]) / (den + exs + jnp.float32(1e-16))
           + hv_ref[...][:ns])
    o_ref[...] = res[:, None]


def _finalize(ns, parts, a_word, w_word, a_sent, hsv):
    npad = a_sent.shape[0]
    return pl.pallas_call(
        functools.partial(_fin_body, ns),
        grid=(1,),
        in_specs=[
            pl.BlockSpec((2 * _NTILES, ns), lambda i: (0, 0)),
            pl.BlockSpec((_BLK,), lambda i: (0,)),
            pl.BlockSpec((_BLK,), lambda i: (0,)),
            pl.BlockSpec((npad,), lambda i: (0,)),
            pl.BlockSpec((npad,), lambda i: (0,)),
        ],
        out_specs=pl.BlockSpec((ns, 1), lambda i: (0, 0)),
        out_shape=jax.ShapeDtypeStruct((ns, 1), jnp.float32),
    )(parts, a_word, w_word, a_sent, hsv)


def kernel(Xw, Xs, E, Erev, W1, b1, W2, b2,
           g1_Ws, g1_Wd, g1_as, g1_ad, g1_b,
           g2_Ws, g2_Wd, g2_as, g2_ad, g2_b,
           W3, b3, W4, b4, W5, b5):
    uw, cw, us, cs = _fold_weights(W1, b1, W2, b2, g2_Ws, g2_Wd, g2_as,
                                   g2_ad, g2_b, W3, b3, W5, b5)
    ns = Xs.shape[0]
    a_word, w_word = _project(Xw, uw, cw)
    a_sent, hsv = _project(Xs, us, cs)
    parts = _edge_softmax(ns, E[1], E[0], a_word, w_word, a_sent)
    return _finalize(ns, parts, a_word, w_word, a_sent, hsv)
